# baseline (speedup 1.0000x reference)
import jax
import jax.numpy as jnp
from jax.experimental import pallas as pl
from jax.experimental.pallas import tpu as pltpu

_ROWS = 1_000_000
_DIM = 64
_BLOCK_ROWS = 20_000


def _l2norm_body(x_ref, o_ref):
    x = x_ref[...]
    ones = jnp.ones((_DIM, _DIM), dtype=jnp.float32)
    n = jax.lax.dot(x * x, ones, preferred_element_type=jnp.float32)
    o_ref[...] = x * jax.lax.rsqrt(n)


def kernel(weight):
    return pl.pallas_call(
        _l2norm_body,
        grid=(_ROWS // _BLOCK_ROWS,),
        in_specs=[pl.BlockSpec((_BLOCK_ROWS, _DIM), lambda i: (i, 0))],
        out_specs=pl.BlockSpec((_BLOCK_ROWS, _DIM), lambda i: (i, 0)),
        out_shape=jax.ShapeDtypeStruct((_ROWS, _DIM), jnp.float32),
        compiler_params=pltpu.CompilerParams(
            dimension_semantics=("parallel",),
        ),
    )(weight)
